# baseline (speedup 1.0000x reference)
import functools

import jax
import jax.numpy as jnp
from jax import lax
from jax.experimental import pallas as pl
from jax.experimental.pallas import tpu as pltpu
from jax.experimental.pallas import tpu_sc as plsc

HIDDEN = 2048
NUM_EXPERTS = 64
TOP_K = 8
ROUTED_SCALING_FACTOR = 1.5

N_TOKENS = 8192
NUM_CHUNKS = 4
CHUNK = N_TOKENS // NUM_CHUNKS
NUM_TILES = 32
TOK_PER_TILE = CHUNK // NUM_TILES
LANES = 16
GROUPS = TOK_PER_TILE // LANES
TC_BLOCK = 256
NEG_INF = float("-inf")
IDX_MASK = (1 << 6) - 1
FIXED_SCALE = float(1 << 24)


def _scores_body(hs_ref, w_ref, bias_ref, sfc_ref, pay_ref):
    hs = hs_ref[...]
    w = w_ref[...]
    bias = bias_ref[...]

    logits = jax.lax.dot_general(
        w, hs, (((1,), (1,)), ((), ())),
        preferred_element_type=jnp.float32)

    m = jnp.max(logits, axis=0, keepdims=True)
    e = jnp.exp(logits - m)
    probs = e / jnp.sum(e, axis=0, keepdims=True)

    eid = jax.lax.broadcasted_iota(jnp.int32, probs.shape, 0)
    fx = (probs * FIXED_SCALE).astype(jnp.int32)
    payload = (fx << 6) | eid

    sfc = probs + bias
    if TOK_PER_TILE >= TC_BLOCK:
        sfc_ref[0] = sfc
        pay_ref[0] = payload
    else:
        for t in range(TC_BLOCK // TOK_PER_TILE):
            lo = t * TOK_PER_TILE
            sfc_ref[t] = sfc[:, lo:lo + TOK_PER_TILE]
            pay_ref[t] = payload[:, lo:lo + TOK_PER_TILE]


@jax.jit
def _tc_scores(hs_chunk, classifier_weight, bias_col):
    if TOK_PER_TILE >= TC_BLOCK:
        blocks_per_tile = TOK_PER_TILE // TC_BLOCK
        out_block = (1, NUM_EXPERTS, TC_BLOCK)

        def omap(i):
            return (i // blocks_per_tile, 0, i % blocks_per_tile)
    else:
        tiles_per_block = TC_BLOCK // TOK_PER_TILE
        out_block = (tiles_per_block, NUM_EXPERTS, TOK_PER_TILE)

        def omap(i):
            return (i, 0, 0)

    return pl.pallas_call(
        _scores_body,
        grid=(CHUNK // TC_BLOCK,),
        in_specs=[
            pl.BlockSpec((TC_BLOCK, HIDDEN), lambda i: (i, 0)),
            pl.BlockSpec((NUM_EXPERTS, HIDDEN), lambda i: (0, 0)),
            pl.BlockSpec((NUM_EXPERTS, 1), lambda i: (0, 0)),
        ],
        out_specs=[
            pl.BlockSpec(out_block, omap),
            pl.BlockSpec(out_block, omap),
        ],
        out_shape=[
            jax.ShapeDtypeStruct(
                (NUM_TILES, NUM_EXPERTS, TOK_PER_TILE), jnp.float32),
            jax.ShapeDtypeStruct(
                (NUM_TILES, NUM_EXPERTS, TOK_PER_TILE), jnp.int32),
        ],
    )(hs_chunk, classifier_weight, bias_col)


def _sc_topk_body(sfc_hbm, pay_hbm, oidx_hbm, owgt_hbm,
                  sfc_v, pay_v, oi_v, ow_v):
    nc = 2
    wid = lax.axis_index("s") * nc + lax.axis_index("c")

    pltpu.sync_copy(sfc_hbm.at[wid], sfc_v)
    pltpu.sync_copy(pay_hbm.at[wid], pay_v)

    for g in range(GROUPS):
        col = g * LANES

        def scan_expert(e, carry):
            vals, pays = carry
            v = sfc_v[e, pl.ds(col, LANES)]
            p = pay_v[e, pl.ds(col, LANES)]
            new_vals = []
            new_pays = []
            for i in range(TOP_K):
                t_v, t_p = vals[i], pays[i]
                c = v > t_v
                new_vals.append(jnp.where(c, v, t_v))
                new_pays.append(jnp.where(c, p, t_p))
                v = jnp.where(c, t_v, v)
                p = jnp.where(c, t_p, p)
            return tuple(new_vals), tuple(new_pays)

        init = (tuple(jnp.full((LANES,), NEG_INF, jnp.float32)
                      for _ in range(TOP_K)),
                tuple(jnp.full((LANES,), 0, jnp.int32)
                      for _ in range(TOP_K)))
        _, pays = lax.fori_loop(0, NUM_EXPERTS, scan_expert, init,
                                unroll=4)

        for i in range(TOP_K):
            p = pays[i]
            e_i = p & IDX_MASK
            w_i = ((p >> 6).astype(jnp.float32)
                   * (ROUTED_SCALING_FACTOR / FIXED_SCALE))
            oi_v[i, pl.ds(col, LANES)] = e_i
            ow_v[i, pl.ds(col, LANES)] = w_i

    pltpu.sync_copy(oi_v, oidx_hbm.at[wid])
    pltpu.sync_copy(ow_v, owgt_hbm.at[wid])


@jax.jit
def _sc_topk(sfc, payload):
    mesh = plsc.VectorSubcoreMesh(core_axis_name="c", subcore_axis_name="s")
    run = functools.partial(
        pl.kernel,
        mesh=mesh,
        out_type=[
            jax.ShapeDtypeStruct((NUM_TILES, TOP_K, TOK_PER_TILE), jnp.int32),
            jax.ShapeDtypeStruct((NUM_TILES, TOP_K, TOK_PER_TILE),
                                 jnp.float32),
        ],
        scratch_types=[
            pltpu.VMEM((NUM_EXPERTS, TOK_PER_TILE), jnp.float32),
            pltpu.VMEM((NUM_EXPERTS, TOK_PER_TILE), jnp.int32),
            pltpu.VMEM((TOP_K, TOK_PER_TILE), jnp.int32),
            pltpu.VMEM((TOP_K, TOK_PER_TILE), jnp.float32),
        ],
    )(_sc_topk_body)
    return run(sfc, payload)


def kernel(hidden_states, classifier_weight, e_score_correction_bias):
    hs = hidden_states.reshape(-1, HIDDEN).astype(jnp.float32)
    bias_col = e_score_correction_bias.reshape(NUM_EXPERTS, 1)

    outs = []
    for c in range(NUM_CHUNKS):
        hs_c = lax.slice_in_dim(hs, c * CHUNK, (c + 1) * CHUNK, axis=0)
        sfc, payload = _tc_scores(hs_c, classifier_weight, bias_col)
        outs.append(_sc_topk(sfc, payload))

    idx = jnp.concatenate(
        [jnp.transpose(i, (0, 2, 1)).reshape(CHUNK, TOP_K) for i, _ in outs])
    wgt = jnp.concatenate(
        [jnp.transpose(w, (0, 2, 1)).reshape(CHUNK, TOP_K) for _, w in outs])
    return idx, wgt
